# Initial kernel scaffold; baseline (speedup 1.0000x reference)
import jax
import jax.numpy as jnp
from jax.experimental import pallas as pl


def kernel(x, eIndex, eAttributes, W1, b1, W2, b2, Wm1, bm1, Wm2, bm2):
    raise NotImplementedError("write your pallas kernel here")


# speedup vs baseline: 2.2827x; 2.2827x over previous
import jax
import jax.numpy as jnp
from jax.experimental import pallas as pl


def kernel(x, eIndex, eAttributes, W1, b1, W2, b2, Wm1, bm1, Wm2, bm2):
    src = eIndex[0].astype(jnp.int32)
    dst = eIndex[1].astype(jnp.int32)
    N = x.shape[0]
    H = W1.shape[1]
    deg = jnp.zeros((N,), jnp.float32).at[dst].add(1.0) + 1.0
    dis = jax.lax.rsqrt(deg)
    u = (x @ W1) * dis[:, None]
    agg = jnp.zeros_like(u).at[dst].add(u[src])
    h = jax.nn.relu((agg + u) * dis[:, None] + b1)
    u = (h @ W2) * dis[:, None]
    agg = jnp.zeros_like(u).at[dst].add(u[src])
    h = (agg + u) * dis[:, None] + b2
    A = h @ Wm1[:H]
    B = h @ Wm1[H:2 * H]
    c = eAttributes @ Wm1[2 * H:] + bm1
    z = jax.nn.relu(A[src] + B[dst] + c)
    return z @ Wm2 + bm2


# speedup vs baseline: 2.7360x; 1.1986x over previous
import functools

import jax
import jax.numpy as jnp
from jax import lax
from jax.experimental import pallas as pl
from jax.experimental.pallas import tpu as pltpu
from jax.experimental.pallas import tpu_sc as plsc

N_NODES = 10000
N_EDGES = 320000
HID = 128
NW = 32
EPW = N_EDGES // NW
CB = 80
NCHUNK = EPW // CB
LANES = 16


def _edge_stage_body(a_hbm, b_hbm, src_hbm, dst_hbm, c_hbm, w_hbm, out_hbm,
                     sidx, didx, rows_a, rows_b, c_v, w_v, o_v, sem):
    wid = lax.axis_index("s") * 2 + lax.axis_index("c")
    base = wid * EPW
    pltpu.sync_copy(w_hbm, w_v)

    @pl.loop(0, NCHUNK)
    def _chunk(i):
        off = base + i * CB
        pltpu.sync_copy(src_hbm.at[pl.ds(off, CB)], sidx)
        pltpu.sync_copy(dst_hbm.at[pl.ds(off, CB)], didx)
        ca = pltpu.async_copy(a_hbm.at[sidx], rows_a, sem)
        cb = pltpu.async_copy(b_hbm.at[didx], rows_b, sem)
        cc = pltpu.async_copy(c_hbm.at[pl.ds(off, CB)], c_v, sem)
        ca.wait()
        cb.wait()
        cc.wait()

        @pl.loop(0, CB)
        def _edge(e):
            acc = jnp.zeros((LANES,), jnp.float32)
            for j in range(HID // LANES):
                sl = pl.ds(j * LANES, LANES)
                g = rows_a[e, sl] + rows_b[e, sl] + c_v[e, sl]
                g = jnp.maximum(g, 0.0)
                acc = acc + g * w_v[sl]
            o_v[e, :] = acc

        pltpu.sync_copy(o_v, out_hbm.at[pl.ds(off, CB)])


@jax.jit
def _edge_stage(A, B, src, dst, c, w):
    mesh = plsc.VectorSubcoreMesh(core_axis_name="c", subcore_axis_name="s")
    k = pl.kernel(
        _edge_stage_body,
        out_type=jax.ShapeDtypeStruct((N_EDGES, LANES), jnp.float32),
        mesh=mesh,
        scratch_types=[
            pltpu.VMEM((CB,), jnp.int32),
            pltpu.VMEM((CB,), jnp.int32),
            pltpu.VMEM((CB, HID), jnp.float32),
            pltpu.VMEM((CB, HID), jnp.float32),
            pltpu.VMEM((CB, HID), jnp.float32),
            pltpu.VMEM((HID,), jnp.float32),
            pltpu.VMEM((CB, LANES), jnp.float32),
            pltpu.SemaphoreType.DMA,
        ],
    )
    return k(A, B, src, dst, c, w)


def kernel(x, eIndex, eAttributes, W1, b1, W2, b2, Wm1, bm1, Wm2, bm2):
    src = eIndex[0].astype(jnp.int32)
    dst = eIndex[1].astype(jnp.int32)
    N = x.shape[0]
    H = W1.shape[1]
    deg = jnp.zeros((N,), jnp.float32).at[dst].add(1.0) + 1.0
    dis = jax.lax.rsqrt(deg)
    u = (x @ W1) * dis[:, None]
    agg = jnp.zeros_like(u).at[dst].add(u[src])
    h = jax.nn.relu((agg + u) * dis[:, None] + b1)
    u = (h @ W2) * dis[:, None]
    agg = jnp.zeros_like(u).at[dst].add(u[src])
    h = (agg + u) * dis[:, None] + b2
    A = h @ Wm1[:H]
    B = h @ Wm1[H:2 * H]
    c = eAttributes @ Wm1[2 * H:] + bm1
    part = _edge_stage(A, B, src, dst, c, Wm2[:, 0])
    return jnp.sum(part, axis=1, keepdims=True) + bm2


# speedup vs baseline: 7.8565x; 2.8715x over previous
import dataclasses
import functools

import jax
import jax.numpy as jnp
from jax import lax
from jax.experimental import pallas as pl
from jax.experimental.pallas import tpu as pltpu
from jax.experimental.pallas import tpu_sc as plsc

N_NODES = 10000
N_EDGES = 320000
HID = 128
NW = 32
EPW = N_EDGES // NW
CB = 80
NCHUNK = EPW // CB
LANES = 16
NPAD = 10240
NPT = NPAD // 16
ZB = 128


def _zero_fill(buf, rows, width):
    @pl.loop(0, rows)
    def _r(r):
        for j in range(width // LANES):
            buf[r, pl.ds(j * LANES, LANES)] = jnp.zeros((LANES,), jnp.float32)


def _deg_body(dst_hbm, out_hbm, didx, cnt_v, sem):
    cid = lax.axis_index("c")
    sid = lax.axis_index("s")
    wid = sid * 2 + cid

    @pl.loop(0, NPAD // LANES)
    def _z(r):
        cnt_v[pl.ds(r * LANES, LANES)] = jnp.zeros((LANES,), jnp.float32)

    base = wid * EPW
    ones = jnp.ones((LANES,), jnp.float32)

    @pl.loop(0, NCHUNK)
    def _chunk(i):
        pltpu.sync_copy(dst_hbm.at[pl.ds(base + i * CB, CB)], didx)

        @pl.loop(0, CB // LANES)
        def _g(g):
            idx = didx[pl.ds(g * LANES, LANES)]
            plsc.addupdate_scatter(cnt_v, [idx], ones)

    pltpu.sync_copy(cnt_v, out_hbm.at[wid])


@jax.jit
def _deg_kernel(dst):
    mesh = plsc.VectorSubcoreMesh(core_axis_name="c", subcore_axis_name="s")
    cp = pltpu.CompilerParams()
    if "needs_layout_passes" in pltpu.CompilerParams.__dataclass_fields__:
        cp = dataclasses.replace(cp, needs_layout_passes=False)
    k = pl.kernel(
        _deg_body,
        out_type=jax.ShapeDtypeStruct((NW, NPAD), jnp.float32),
        mesh=mesh,
        compiler_params=cp,
        scratch_types=[
            pltpu.VMEM((CB,), jnp.int32),
            pltpu.VMEM((NPAD,), jnp.float32),
            pltpu.SemaphoreType.DMA,
        ],
    )
    return k(dst)


def _agg_body(u_hbm, src_hbm, dst_hbm, out_hbm,
              sidx, didx, rows_v, zero_v, acc_sh, sem):
    cid = lax.axis_index("c")
    sid = lax.axis_index("s")
    wid = sid * 2 + cid

    _zero_fill(zero_v, ZB, HID)
    for t in range(NPT // ZB):
        pltpu.sync_copy(zero_v, acc_sh.at[pl.ds(sid * NPT + t * ZB, ZB)])
    plsc.subcore_barrier()

    base = wid * EPW

    @pl.loop(0, NCHUNK)
    def _chunk(i):
        off = base + i * CB
        pltpu.sync_copy(src_hbm.at[pl.ds(off, CB)], sidx)
        pltpu.sync_copy(dst_hbm.at[pl.ds(off, CB)], didx)
        pltpu.async_copy(u_hbm.at[sidx], rows_v, sem).wait()
        pltpu.sync_copy(rows_v, acc_sh.at[didx], add=True)

    plsc.subcore_barrier()
    row0 = cid * NPAD + sid * NPT
    pltpu.sync_copy(acc_sh.at[pl.ds(sid * NPT, NPT)],
                    out_hbm.at[pl.ds(row0, NPT)])


@jax.jit
def _agg_kernel(u, src, dst):
    mesh = plsc.VectorSubcoreMesh(core_axis_name="c", subcore_axis_name="s")
    k = pl.kernel(
        _agg_body,
        out_type=jax.ShapeDtypeStruct((2 * NPAD, HID), jnp.float32),
        mesh=mesh,
        scratch_types=[
            pltpu.VMEM((CB,), jnp.int32),
            pltpu.VMEM((CB,), jnp.int32),
            pltpu.VMEM((CB, HID), jnp.float32),
            pltpu.VMEM((ZB, HID), jnp.float32),
            pltpu.VMEM_SHARED((NPAD, HID), jnp.float32),
            pltpu.SemaphoreType.DMA,
        ],
    )
    return k(u, src, dst)


def _edge_stage_body(a_hbm, b_hbm, src_hbm, dst_hbm, c_hbm, w_hbm, out_hbm,
                     sidx, didx, rows_a, rows_b, c_v, w_v, o_v, sem):
    wid = lax.axis_index("s") * 2 + lax.axis_index("c")
    base = wid * EPW
    pltpu.sync_copy(w_hbm, w_v)

    @pl.loop(0, NCHUNK)
    def _chunk(i):
        off = base + i * CB
        pltpu.sync_copy(src_hbm.at[pl.ds(off, CB)], sidx)
        pltpu.sync_copy(dst_hbm.at[pl.ds(off, CB)], didx)
        ca = pltpu.async_copy(a_hbm.at[sidx], rows_a, sem)
        cb = pltpu.async_copy(b_hbm.at[didx], rows_b, sem)
        cc = pltpu.async_copy(c_hbm.at[pl.ds(off, CB)], c_v, sem)
        ca.wait()
        cb.wait()
        cc.wait()

        @pl.loop(0, CB)
        def _edge(e):
            acc = jnp.zeros((LANES,), jnp.float32)
            for j in range(HID // LANES):
                sl = pl.ds(j * LANES, LANES)
                g = rows_a[e, sl] + rows_b[e, sl] + c_v[e, sl]
                g = jnp.maximum(g, 0.0)
                acc = acc + g * w_v[sl]
            o_v[e, :] = acc

        pltpu.sync_copy(o_v, out_hbm.at[pl.ds(off, CB)])


@jax.jit
def _edge_stage(A, B, src, dst, c, w):
    mesh = plsc.VectorSubcoreMesh(core_axis_name="c", subcore_axis_name="s")
    k = pl.kernel(
        _edge_stage_body,
        out_type=jax.ShapeDtypeStruct((N_EDGES, LANES), jnp.float32),
        mesh=mesh,
        scratch_types=[
            pltpu.VMEM((CB,), jnp.int32),
            pltpu.VMEM((CB,), jnp.int32),
            pltpu.VMEM((CB, HID), jnp.float32),
            pltpu.VMEM((CB, HID), jnp.float32),
            pltpu.VMEM((CB, HID), jnp.float32),
            pltpu.VMEM((HID,), jnp.float32),
            pltpu.VMEM((CB, LANES), jnp.float32),
            pltpu.SemaphoreType.DMA,
        ],
    )
    return k(A, B, src, dst, c, w)


def kernel(x, eIndex, eAttributes, W1, b1, W2, b2, Wm1, bm1, Wm2, bm2):
    src = eIndex[0].astype(jnp.int32)
    dst = eIndex[1].astype(jnp.int32)
    H = W1.shape[1]

    degp = _deg_kernel(dst)
    deg = jnp.sum(degp, axis=0)[:N_NODES] + 1.0
    dis = jax.lax.rsqrt(deg)

    u = (x @ W1) * dis[:, None]
    aggp = _agg_kernel(u, src, dst)
    agg = aggp[:N_NODES] + aggp[NPAD:NPAD + N_NODES]
    h = jax.nn.relu((agg + u) * dis[:, None] + b1)
    u = (h @ W2) * dis[:, None]
    aggp = _agg_kernel(u, src, dst)
    agg = aggp[:N_NODES] + aggp[NPAD:NPAD + N_NODES]
    h = (agg + u) * dis[:, None] + b2

    A = h @ Wm1[:H]
    B = h @ Wm1[H:2 * H]
    c = eAttributes @ Wm1[2 * H:] + bm1
    part = _edge_stage(A, B, src, dst, c, Wm2[:, 0])
    return jnp.sum(part, axis=1, keepdims=True) + bm2
